# Initial kernel scaffold; baseline (speedup 1.0000x reference)
import jax
import jax.numpy as jnp
from jax.experimental import pallas as pl


def kernel(x, W, input_ids):
    raise NotImplementedError("write your pallas kernel here")


# speedup vs baseline: 81.2701x; 81.2701x over previous
import functools

import jax
import jax.numpy as jnp
from jax import lax
from jax.experimental import pallas as pl
from jax.experimental.pallas import tpu as pltpu
from jax.experimental.pallas import tpu_sc as plsc

NUM_INPUTS = 512
NUM_COMPUTED = 4096
NUM_OUTPUTS = 128
FAN_IN = 64
SCALE = 4.9
N_UNITS = NUM_INPUTS + 1 + NUM_COMPUTED
CARRY_PAD = 4624
CHUNK = 512
N_CHUNKS = NUM_COMPUTED // CHUNK
OUT_BASE = NUM_INPUTS + 1 + (NUM_COMPUTED - NUM_OUTPUTS)


def _body(x_hbm, w_hbm, idx_hbm, out_hbm, carry, w_v, idx_v, st):
    wid = lax.axis_index("s") * 2 + lax.axis_index("c")

    @pl.when(wid == 0)
    def _():
        lane = jnp.arange(16, dtype=jnp.int32)
        ones = jnp.ones((16,), jnp.float32)

        pltpu.sync_copy(x_hbm, carry.at[pl.ds(0, NUM_INPUTS)])

        def init_ones(i, _):
            carry[pl.ds(NUM_INPUTS + 16 * i, 16)] = ones
            return _

        lax.fori_loop(0, (CARRY_PAD - NUM_INPUTS) // 16, init_ones, 0)

        def unit_step(u, pos):
            base = u * FAN_IN
            acc = jnp.zeros((16,), jnp.float32)
            for k in range(FAN_IN // 16):
                iv = idx_v[pl.ds(base + 16 * k, 16)]
                wv = w_v[pl.ds(base + 16 * k, 16)]
                vals = plsc.load_gather(carry, [iv])
                acc = acc + vals * wv
            s = jnp.sum(acc)
            sv = jnp.full((16,), SCALE, jnp.float32) * s
            val = 1.0 / (1.0 + jnp.exp(-sv))
            posv = jnp.full((16,), 0, jnp.int32) + pos
            plsc.store_scatter(carry, [posv], val, mask=lane == 0)
            return pos + 1

        def chunk_step(c, pos):
            off = c * (CHUNK * FAN_IN)
            pltpu.sync_copy(w_hbm.at[pl.ds(off, CHUNK * FAN_IN)], w_v)
            pltpu.sync_copy(idx_hbm.at[pl.ds(off, CHUNK * FAN_IN)], idx_v)
            return lax.fori_loop(0, CHUNK, unit_step, pos)

        lax.fori_loop(0, N_CHUNKS, chunk_step, NUM_INPUTS + 1)

        for i in range(NUM_OUTPUTS // 16):
            iv = jnp.full((16,), OUT_BASE + 16 * i, jnp.int32) + lane
            st[pl.ds(16 * i, 16)] = plsc.load_gather(carry, [iv])
        pltpu.sync_copy(st, out_hbm)


@jax.jit
def kernel(x, W, input_ids):
    mesh = plsc.VectorSubcoreMesh(core_axis_name="c", subcore_axis_name="s")
    run = pl.kernel(
        _body,
        out_type=jax.ShapeDtypeStruct((NUM_OUTPUTS,), jnp.float32),
        mesh=mesh,
        scratch_types=[
            pltpu.VMEM((CARRY_PAD,), jnp.float32),
            pltpu.VMEM((CHUNK * FAN_IN,), jnp.float32),
            pltpu.VMEM((CHUNK * FAN_IN,), jnp.int32),
            pltpu.VMEM((NUM_OUTPUTS,), jnp.float32),
        ],
        compiler_params=pltpu.CompilerParams(needs_layout_passes=False),
    )
    out = run(x.reshape(-1), W.reshape(-1), input_ids.reshape(-1))
    return out[None, :]


# speedup vs baseline: 82.1634x; 1.0110x over previous
import jax
import jax.numpy as jnp
from jax import lax
from jax.experimental import pallas as pl
from jax.experimental.pallas import tpu as pltpu
from jax.experimental.pallas import tpu_sc as plsc

NUM_INPUTS = 512
NUM_COMPUTED = 4096
NUM_OUTPUTS = 128
FAN_IN = 64
SCALE = 4.9
N_UNITS = NUM_INPUTS + 1 + NUM_COMPUTED
CARRY_PAD = 4624
CHUNK = 512
N_CHUNKS = NUM_COMPUTED // CHUNK
GROUPS = CHUNK // 16
OUT_BASE = NUM_INPUTS + 1 + (NUM_COMPUTED - NUM_OUTPUTS)


def _body(x_hbm, w_hbm, idx_hbm, out_hbm, carry, w_v, idx_v, st):
    wid = lax.axis_index("s") * 2 + lax.axis_index("c")

    @pl.when(wid == 0)
    def _():
        lane = jnp.arange(16, dtype=jnp.int32)
        ones = jnp.ones((16,), jnp.float32)

        pltpu.sync_copy(x_hbm, carry.at[pl.ds(0, NUM_INPUTS)])

        def init_ones(i, _):
            carry[pl.ds(NUM_INPUTS + 16 * i, 16)] = ones
            return _

        lax.fori_loop(0, (CARRY_PAD - NUM_INPUTS) // 16, init_ones, 0)

        def group_pass(goff, posv, base_pos, want_cnt):
            acc = jnp.zeros((16,), jnp.float32)
            cnt = jnp.zeros((16,), jnp.int32)
            for k in range(FAN_IN):
                iv = idx_v[pl.ds(goff + 16 * k, 16)]
                wv = w_v[pl.ds(goff + 16 * k, 16)]
                vals = plsc.load_gather(carry, [iv])
                vals = jnp.where(iv < posv, vals, 1.0)
                acc = acc + vals * wv
                if want_cnt:
                    internal = jnp.logical_and(iv >= base_pos, iv < posv)
                    cnt = cnt + internal.astype(jnp.int32)
            val = 1.0 / (1.0 + jnp.exp(-SCALE * acc))
            return val, cnt

        def group_step(g, pos):
            goff = g * (16 * FAN_IN)
            posv = pos + lane
            val, cnt = group_pass(goff, posv, pos, True)
            plsc.store_scatter(carry, [posv], val)
            n_int = jnp.sum(cnt)

            def fix_body(d):
                vcur = plsc.load_gather(carry, [posv])
                vnew, _ = group_pass(goff, posv, pos, False)
                plsc.store_scatter(carry, [posv], vnew)
                return jnp.sum((vnew != vcur).astype(jnp.int32))

            lax.while_loop(lambda d: d > 0, fix_body, n_int)
            return pos + 16

        def chunk_step(c, pos):
            off = c * (CHUNK * FAN_IN)
            pltpu.sync_copy(w_hbm.at[pl.ds(off, CHUNK * FAN_IN)], w_v)
            pltpu.sync_copy(idx_hbm.at[pl.ds(off, CHUNK * FAN_IN)], idx_v)
            return lax.fori_loop(0, GROUPS, group_step, pos)

        lax.fori_loop(0, N_CHUNKS, chunk_step, NUM_INPUTS + 1)

        for i in range(NUM_OUTPUTS // 16):
            iv = jnp.full((16,), OUT_BASE + 16 * i, jnp.int32) + lane
            st[pl.ds(16 * i, 16)] = plsc.load_gather(carry, [iv])
        pltpu.sync_copy(st, out_hbm)


@jax.jit
def kernel(x, W, input_ids):
    mesh = plsc.VectorSubcoreMesh(core_axis_name="c", subcore_axis_name="s")
    run = pl.kernel(
        _body,
        out_type=jax.ShapeDtypeStruct((NUM_OUTPUTS,), jnp.float32),
        mesh=mesh,
        scratch_types=[
            pltpu.VMEM((CARRY_PAD,), jnp.float32),
            pltpu.VMEM((CHUNK * FAN_IN,), jnp.float32),
            pltpu.VMEM((CHUNK * FAN_IN,), jnp.int32),
            pltpu.VMEM((NUM_OUTPUTS,), jnp.float32),
        ],
        compiler_params=pltpu.CompilerParams(needs_layout_passes=False),
    )
    wT = W.reshape(-1, 16, FAN_IN).transpose(0, 2, 1).reshape(-1)
    idxT = input_ids.reshape(-1, 16, FAN_IN).transpose(0, 2, 1).reshape(-1)
    out = run(x.reshape(-1), wT, idxT)
    return out[None, :]
